# baseline (speedup 1.0000x reference)
import functools

import jax
import jax.numpy as jnp
from jax import lax
from jax.experimental import pallas as pl
from jax.experimental.pallas import tpu as pltpu
from jax.experimental.pallas import tpu_sc as plsc

B = 4096
HIST = 50
EMB = 64
ROW = EMB + 1
ROWP = 72
SLOTS = 2 + HIST
UI_LEN = 200001
NC = 2
NS = 16
NW = NC * NS
BPW = B // NW
CHUNK = 16
NCHUNK = BPW // CHUNK
CROWS = CHUNK * SLOTS


def _build_sc_gather():
    mesh = plsc.VectorSubcoreMesh(core_axis_name="c", subcore_axis_name="s",
                                  num_cores=NC, num_subcores=NS)

    @functools.partial(
        pl.kernel,
        mesh=mesh,
        compiler_params=pltpu.CompilerParams(needs_layout_passes=False,
                                             use_tc_tiling_on_sc=False),
        out_type=(
            jax.ShapeDtypeStruct((B * SLOTS, EMB), jnp.float32),
            jax.ShapeDtypeStruct((B * SLOTS, 1), jnp.float32),
        ),
        scratch_types=[
            pltpu.VMEM((CROWS, ROWP), jnp.float32),
            pltpu.VMEM((CROWS, ROWP), jnp.float32),
            pltpu.VMEM((CROWS,), jnp.int32),
            pltpu.VMEM((CROWS,), jnp.int32),
            pltpu.SemaphoreType.DMA,
            pltpu.SemaphoreType.DMA,
        ],
    )
    def sc_gather(idx_all, table, nz_out, biascol_out,
                  buf0, buf1, idx0, idx1, gsem, osem):
        wid = lax.axis_index("s") * NC + lax.axis_index("c")
        r0 = wid * BPW * SLOTS
        bufs = (buf0, buf1)
        idxs = (idx0, idx1)

        def stage_and_fire(k):
            buf, idx = bufs[k % 2], idxs[k % 2]
            pltpu.sync_copy(idx_all.at[pl.ds(r0 + k * CROWS, CROWS)], idx)
            pltpu.async_copy(table.at[idx], buf, gsem)

        def write_out(k):
            buf = bufs[k % 2]
            base = r0 + k * CROWS
            pltpu.async_copy(buf.at[:, pl.ds(0, EMB)],
                             nz_out.at[pl.ds(base, CROWS), :], osem)
            pltpu.async_copy(buf.at[:, pl.ds(EMB, 1)],
                             biascol_out.at[pl.ds(base, CROWS), :], osem)

        def drain_gather(k):
            buf, idx = bufs[k % 2], idxs[k % 2]
            pltpu.make_async_copy(table.at[idx], buf, gsem).wait()

        def drain_out(k):
            buf = bufs[k % 2]
            base = r0 + k * CROWS
            pltpu.make_async_copy(
                buf.at[:, pl.ds(0, EMB)],
                nz_out.at[pl.ds(base, CROWS), :], osem).wait()
            pltpu.make_async_copy(
                buf.at[:, pl.ds(EMB, 1)],
                biascol_out.at[pl.ds(base, CROWS), :], osem).wait()

        stage_and_fire(0)
        for k in range(NCHUNK):
            if k + 1 < NCHUNK:
                if k >= 1:
                    drain_out(k - 1)
                stage_and_fire(k + 1)
            drain_gather(k)
            write_out(k)
        drain_out(NCHUNK - 2)
        drain_out(NCHUNK - 1)

    return sc_gather


_SC_GATHER = _build_sc_gather()

TC_BLOCK = 512


def _tc_reduce_body(nz_ref, bias_ref, out_ref):
    nz = nz_ref[...]
    u1 = nz[:, 0, :]
    u2 = nz[:, 1, :]
    s2 = jnp.sum(nz[:, 2:, :], axis=1)
    fm = u1 * u2 + (u1 + u2) * s2
    out_ref[...] = (jnp.sum(fm, axis=1, keepdims=True)
                    + bias_ref[0])


def _tc_reduce(nz, bias):
    return pl.pallas_call(
        _tc_reduce_body,
        grid=(B // TC_BLOCK,),
        in_specs=[
            pl.BlockSpec((TC_BLOCK, SLOTS, EMB), lambda i: (i, 0, 0)),
            pl.BlockSpec(memory_space=pltpu.SMEM),
        ],
        out_specs=pl.BlockSpec((TC_BLOCK, 1), lambda i: (i, 0)),
        out_shape=jax.ShapeDtypeStruct((B, 1), jnp.float32),
    )(nz, bias)


@jax.jit
def _fm(ui_pair, preference_index, ui_emb_w, feature_emb_w, bias):
    ui_p = jnp.pad(ui_emb_w, ((0, 0), (0, ROWP - ROW)))
    feat_p = jnp.pad(feature_emb_w, ((0, 0), (0, ROWP - ROW)))
    table = jnp.concatenate([ui_p, feat_p], axis=0)
    idx_all = jnp.concatenate(
        [ui_pair.astype(jnp.int32),
         preference_index.astype(jnp.int32) + UI_LEN], axis=1)
    nzflat, biasflat = _SC_GATHER(idx_all.reshape(-1), table)
    nz = nzflat.reshape(B, SLOTS, EMB)
    result = _tc_reduce(nz, bias)
    return result, biasflat.reshape(B, SLOTS, 1), nz


def kernel(ui_pair, feature_index, preference_index, ui_emb_w, feature_emb_w,
           bias):
    del feature_index
    return _fm(ui_pair, preference_index, ui_emb_w, feature_emb_w, bias)


# speedup vs baseline: 2.2144x; 2.2144x over previous
import functools

import jax
import jax.numpy as jnp
from jax import lax
from jax.experimental import pallas as pl
from jax.experimental.pallas import tpu as pltpu
from jax.experimental.pallas import tpu_sc as plsc

B = 4096
HIST = 50
EMB = 64
ROW = EMB + 1
ROWP = 128
SLOTS = 2 + HIST
UI_LEN = 200001
NC = 2
NS = 16
NW = NC * NS
BPW = B // NW
CHUNK = 8
NCHUNK = BPW // CHUNK
CROWS = CHUNK * SLOTS


def _build_sc_gather():
    mesh = plsc.VectorSubcoreMesh(core_axis_name="c", subcore_axis_name="s",
                                  num_cores=NC, num_subcores=NS)

    @functools.partial(
        pl.kernel,
        mesh=mesh,
        out_type=jax.ShapeDtypeStruct((B * SLOTS, ROWP), jnp.float32),
        scratch_types=[
            pltpu.VMEM((CROWS, ROWP), jnp.float32),
            pltpu.VMEM((CROWS, ROWP), jnp.float32),
            pltpu.VMEM((CROWS,), jnp.int32),
            pltpu.VMEM((CROWS,), jnp.int32),
            pltpu.SemaphoreType.DMA,
            pltpu.SemaphoreType.DMA,
        ],
    )
    def sc_gather(idx_all, table, raw_out,
                  buf0, buf1, idx0, idx1, gsem, osem):
        wid = lax.axis_index("s") * NC + lax.axis_index("c")
        r0 = wid * BPW * SLOTS
        bufs = (buf0, buf1)
        idxs = (idx0, idx1)

        def stage_and_fire(k):
            buf, idx = bufs[k % 2], idxs[k % 2]
            pltpu.sync_copy(idx_all.at[pl.ds(r0 + k * CROWS, CROWS)], idx)
            pltpu.async_copy(table.at[idx], buf, gsem)

        def drain_gather(k):
            buf, idx = bufs[k % 2], idxs[k % 2]
            pltpu.make_async_copy(table.at[idx], buf, gsem).wait()

        def write_out(k):
            pltpu.async_copy(bufs[k % 2],
                             raw_out.at[pl.ds(r0 + k * CROWS, CROWS), :],
                             osem)

        def drain_out(k):
            pltpu.make_async_copy(
                bufs[k % 2],
                raw_out.at[pl.ds(r0 + k * CROWS, CROWS), :], osem).wait()

        stage_and_fire(0)
        for k in range(NCHUNK):
            if k + 1 < NCHUNK:
                if k >= 1:
                    drain_out(k - 1)
                stage_and_fire(k + 1)
            drain_gather(k)
            write_out(k)
        drain_out(NCHUNK - 2)
        drain_out(NCHUNK - 1)

    return sc_gather


_SC_GATHER = _build_sc_gather()

TCB = 128


def _tc_split_body(raw_ref, bias_ref, nz_ref, biascol_ref, res_ref):
    raw = raw_ref[...]
    nz = raw[:, :EMB].reshape(TCB, SLOTS, EMB)
    nz_ref[...] = nz
    biascol_ref[...] = raw[:, EMB:EMB + 1].reshape(TCB, SLOTS, 1)
    u1 = nz[:, 0, :]
    u2 = nz[:, 1, :]
    s2 = jnp.sum(nz[:, 2:, :], axis=1)
    fm = u1 * u2 + (u1 + u2) * s2
    res_ref[...] = jnp.sum(fm, axis=1, keepdims=True) + bias_ref[0]


def _tc_split(raw, bias):
    return pl.pallas_call(
        _tc_split_body,
        grid=(B // TCB,),
        in_specs=[
            pl.BlockSpec((TCB * SLOTS, ROWP), lambda i: (i, 0)),
            pl.BlockSpec(memory_space=pltpu.SMEM),
        ],
        out_specs=[
            pl.BlockSpec((TCB, SLOTS, EMB), lambda i: (i, 0, 0)),
            pl.BlockSpec((TCB, SLOTS, 1), lambda i: (i, 0, 0)),
            pl.BlockSpec((TCB, 1), lambda i: (i, 0)),
        ],
        out_shape=(
            jax.ShapeDtypeStruct((B, SLOTS, EMB), jnp.float32),
            jax.ShapeDtypeStruct((B, SLOTS, 1), jnp.float32),
            jax.ShapeDtypeStruct((B, 1), jnp.float32),
        ),
    )(raw, bias)


@jax.jit
def _fm(ui_pair, preference_index, ui_emb_w, feature_emb_w, bias):
    ui_p = jnp.pad(ui_emb_w, ((0, 0), (0, ROWP - ROW)))
    feat_p = jnp.pad(feature_emb_w, ((0, 0), (0, ROWP - ROW)))
    table = jnp.concatenate([ui_p, feat_p], axis=0)
    idx_all = jnp.concatenate(
        [ui_pair.astype(jnp.int32),
         preference_index.astype(jnp.int32) + UI_LEN], axis=1)
    raw = _SC_GATHER(idx_all.reshape(-1), table)
    nz, biascol, result = _tc_split(raw, bias)
    return result, biascol, nz


def kernel(ui_pair, feature_index, preference_index, ui_emb_w, feature_emb_w,
           bias):
    del feature_index
    return _fm(ui_pair, preference_index, ui_emb_w, feature_emb_w, bias)


# speedup vs baseline: 2.8214x; 1.2741x over previous
import functools

import jax
import jax.numpy as jnp
from jax import lax
from jax.experimental import pallas as pl
from jax.experimental.pallas import tpu as pltpu
from jax.experimental.pallas import tpu_sc as plsc

B = 4096
HIST = 50
EMB = 64
ROW = EMB + 1
ROWP = 128
SLOTS = 2 + HIST
NC = 2
NS = 16
NW = NC * NS
BPW = B // NW
CHUNK = 8
NCHUNK = BPW // CHUNK
UROWS = 2 * CHUNK
PROWS = HIST * CHUNK

PAD_BLOCK = 2048


def _pad_body(in_ref, out_ref):
    out_ref[...] = jnp.pad(in_ref[...], ((0, 0), (0, ROWP - ROW)))


def _pad_table(t):
    rows = t.shape[0]
    return pl.pallas_call(
        _pad_body,
        grid=(pl.cdiv(rows, PAD_BLOCK),),
        in_specs=[pl.BlockSpec((PAD_BLOCK, ROW), lambda i: (i, 0))],
        out_specs=pl.BlockSpec((PAD_BLOCK, ROWP), lambda i: (i, 0)),
        out_shape=jax.ShapeDtypeStruct((rows, ROWP), jnp.float32),
    )(t)


def _build_sc_gather():
    mesh = plsc.VectorSubcoreMesh(core_axis_name="c", subcore_axis_name="s",
                                  num_cores=NC, num_subcores=NS)

    @functools.partial(
        pl.kernel,
        mesh=mesh,
        out_type=(
            jax.ShapeDtypeStruct((B * 2, ROWP), jnp.float32),
            jax.ShapeDtypeStruct((B * HIST, ROWP), jnp.float32),
        ),
        scratch_types=[
            pltpu.VMEM((UROWS, ROWP), jnp.float32),
            pltpu.VMEM((UROWS, ROWP), jnp.float32),
            pltpu.VMEM((PROWS, ROWP), jnp.float32),
            pltpu.VMEM((PROWS, ROWP), jnp.float32),
            pltpu.VMEM((UROWS,), jnp.int32),
            pltpu.VMEM((UROWS,), jnp.int32),
            pltpu.VMEM((PROWS,), jnp.int32),
            pltpu.VMEM((PROWS,), jnp.int32),
            pltpu.SemaphoreType.DMA,
            pltpu.SemaphoreType.DMA,
        ],
    )
    def sc_gather(ui_idx, pref_idx, ui_w, feat_w, ui_out, pref_out,
                  ubuf0, ubuf1, pbuf0, pbuf1, uidx0, uidx1, pidx0, pidx1,
                  gsem, osem):
        wid = lax.axis_index("s") * NC + lax.axis_index("c")
        u0 = wid * BPW * 2
        p0 = wid * BPW * HIST
        ubufs, pbufs = (ubuf0, ubuf1), (pbuf0, pbuf1)
        uidxs, pidxs = (uidx0, uidx1), (pidx0, pidx1)

        def stage_and_fire(k):
            ubuf, pbuf = ubufs[k % 2], pbufs[k % 2]
            uidx, pidx = uidxs[k % 2], pidxs[k % 2]
            pltpu.sync_copy(ui_idx.at[pl.ds(u0 + k * UROWS, UROWS)], uidx)
            pltpu.sync_copy(pref_idx.at[pl.ds(p0 + k * PROWS, PROWS)], pidx)
            pltpu.async_copy(ui_w.at[uidx], ubuf, gsem)
            pltpu.async_copy(feat_w.at[pidx], pbuf, gsem)

        def drain_gather(k):
            pltpu.make_async_copy(ui_w.at[uidxs[k % 2]], ubufs[k % 2],
                                  gsem).wait()
            pltpu.make_async_copy(feat_w.at[pidxs[k % 2]], pbufs[k % 2],
                                  gsem).wait()

        def write_out(k):
            pltpu.async_copy(ubufs[k % 2],
                             ui_out.at[pl.ds(u0 + k * UROWS, UROWS), :], osem)
            pltpu.async_copy(pbufs[k % 2],
                             pref_out.at[pl.ds(p0 + k * PROWS, PROWS), :],
                             osem)

        def drain_out(k):
            pltpu.make_async_copy(
                ubufs[k % 2],
                ui_out.at[pl.ds(u0 + k * UROWS, UROWS), :], osem).wait()
            pltpu.make_async_copy(
                pbufs[k % 2],
                pref_out.at[pl.ds(p0 + k * PROWS, PROWS), :], osem).wait()

        stage_and_fire(0)
        for k in range(NCHUNK):
            if k + 1 < NCHUNK:
                if k >= 1:
                    drain_out(k - 1)
                stage_and_fire(k + 1)
            drain_gather(k)
            write_out(k)
        drain_out(NCHUNK - 2)
        drain_out(NCHUNK - 1)

    return sc_gather


_SC_GATHER = _build_sc_gather()

TCB = 128


def _tc_split_body(ui_ref, pref_ref, bias_ref, nz_ref, biascol_ref, res_ref):
    ui = ui_ref[...]
    pref = pref_ref[...]
    ui_nz = ui[:, :EMB].reshape(TCB, 2, EMB)
    pref_nz = pref[:, :EMB].reshape(TCB, HIST, EMB)
    nz_ref[...] = jnp.concatenate([ui_nz, pref_nz], axis=1)
    biascol_ref[...] = jnp.concatenate(
        [ui[:, EMB:EMB + 1].reshape(TCB, 2, 1),
         pref[:, EMB:EMB + 1].reshape(TCB, HIST, 1)], axis=1)
    u1 = ui_nz[:, 0, :]
    u2 = ui_nz[:, 1, :]
    s2 = jnp.sum(pref_nz, axis=1)
    fm = u1 * u2 + (u1 + u2) * s2
    res_ref[...] = jnp.sum(fm, axis=1, keepdims=True) + bias_ref[0]


def _tc_split(raw_ui, raw_pref, bias):
    return pl.pallas_call(
        _tc_split_body,
        grid=(B // TCB,),
        in_specs=[
            pl.BlockSpec((TCB * 2, ROWP), lambda i: (i, 0)),
            pl.BlockSpec((TCB * HIST, ROWP), lambda i: (i, 0)),
            pl.BlockSpec(memory_space=pltpu.SMEM),
        ],
        out_specs=[
            pl.BlockSpec((TCB, SLOTS, EMB), lambda i: (i, 0, 0)),
            pl.BlockSpec((TCB, SLOTS, 1), lambda i: (i, 0, 0)),
            pl.BlockSpec((TCB, 1), lambda i: (i, 0)),
        ],
        out_shape=(
            jax.ShapeDtypeStruct((B, SLOTS, EMB), jnp.float32),
            jax.ShapeDtypeStruct((B, SLOTS, 1), jnp.float32),
            jax.ShapeDtypeStruct((B, 1), jnp.float32),
        ),
    )(raw_ui, raw_pref, bias)


@jax.jit
def _fm(ui_pair, preference_index, ui_emb_w, feature_emb_w, bias):
    ui128 = _pad_table(ui_emb_w)
    feat128 = _pad_table(feature_emb_w)
    raw_ui, raw_pref = _SC_GATHER(
        ui_pair.astype(jnp.int32).reshape(-1),
        preference_index.astype(jnp.int32).reshape(-1),
        ui128, feat128)
    nz, biascol, result = _tc_split(raw_ui, raw_pref, bias)
    return result, biascol, nz


def kernel(ui_pair, feature_index, preference_index, ui_emb_w, feature_emb_w,
           bias):
    del feature_index
    return _fm(ui_pair, preference_index, ui_emb_w, feature_emb_w, bias)


# speedup vs baseline: 2.8924x; 1.0251x over previous
import functools

import jax
import jax.numpy as jnp
from jax import lax
from jax.experimental import pallas as pl
from jax.experimental.pallas import tpu as pltpu
from jax.experimental.pallas import tpu_sc as plsc

B = 4096
HIST = 50
EMB = 64
ROW = EMB + 1
ROWP = 128
SLOTS = 2 + HIST
LANES = 16
NC = 2
NS = 16
NW = NC * NS
BPW = B // NW
CHUNK = 8
NCHUNK = BPW // CHUNK
UROWS = 2 * CHUNK
PROWS = HIST * CHUNK

PAD_BLOCK = 2048


def _pad_body(in_ref, out_ref):
    out_ref[...] = jnp.pad(in_ref[...], ((0, 0), (0, ROWP - ROW)))


def _pad_table(t):
    rows = t.shape[0]
    return pl.pallas_call(
        _pad_body,
        grid=(pl.cdiv(rows, PAD_BLOCK),),
        in_specs=[pl.BlockSpec((PAD_BLOCK, ROW), lambda i: (i, 0))],
        out_specs=pl.BlockSpec((PAD_BLOCK, ROWP), lambda i: (i, 0)),
        out_shape=jax.ShapeDtypeStruct((rows, ROWP), jnp.float32),
    )(t)


def _build_sc_gather():
    mesh = plsc.VectorSubcoreMesh(core_axis_name="c", subcore_axis_name="s",
                                  num_cores=NC, num_subcores=NS)

    @functools.partial(
        pl.kernel,
        mesh=mesh,
        out_type=(
            jax.ShapeDtypeStruct((B * 2, ROWP), jnp.float32),
            jax.ShapeDtypeStruct((B * HIST, ROWP), jnp.float32),
            jax.ShapeDtypeStruct((B, EMB), jnp.float32),
        ),
        scratch_types=[
            pltpu.VMEM((UROWS, ROWP), jnp.float32),
            pltpu.VMEM((UROWS, ROWP), jnp.float32),
            pltpu.VMEM((PROWS, ROWP), jnp.float32),
            pltpu.VMEM((PROWS, ROWP), jnp.float32),
            pltpu.VMEM((CHUNK, EMB), jnp.float32),
            pltpu.VMEM((CHUNK, EMB), jnp.float32),
            pltpu.VMEM((UROWS,), jnp.int32),
            pltpu.VMEM((UROWS,), jnp.int32),
            pltpu.VMEM((PROWS,), jnp.int32),
            pltpu.VMEM((PROWS,), jnp.int32),
            pltpu.SemaphoreType.DMA,
            pltpu.SemaphoreType.DMA,
        ],
    )
    def sc_gather(ui_idx, pref_idx, ui_w, feat_w, ui_out, pref_out, s2_out,
                  ubuf0, ubuf1, pbuf0, pbuf1, s2v0, s2v1,
                  uidx0, uidx1, pidx0, pidx1, gsem, osem):
        wid = lax.axis_index("s") * NC + lax.axis_index("c")
        u0 = wid * BPW * 2
        p0 = wid * BPW * HIST
        b0 = wid * BPW
        ubufs, pbufs = (ubuf0, ubuf1), (pbuf0, pbuf1)
        s2vs = (s2v0, s2v1)
        uidxs, pidxs = (uidx0, uidx1), (pidx0, pidx1)

        def stage_and_fire(k):
            ubuf, pbuf = ubufs[k % 2], pbufs[k % 2]
            uidx, pidx = uidxs[k % 2], pidxs[k % 2]
            pltpu.sync_copy(ui_idx.at[pl.ds(u0 + k * UROWS, UROWS)], uidx)
            pltpu.sync_copy(pref_idx.at[pl.ds(p0 + k * PROWS, PROWS)], pidx)
            pltpu.async_copy(ui_w.at[uidx], ubuf, gsem)
            pltpu.async_copy(feat_w.at[pidx], pbuf, gsem)

        def drain_gather(k):
            pltpu.make_async_copy(ui_w.at[uidxs[k % 2]], ubufs[k % 2],
                                  gsem).wait()
            pltpu.make_async_copy(feat_w.at[pidxs[k % 2]], pbufs[k % 2],
                                  gsem).wait()

        def compute_s2(k):
            pbuf, s2v = pbufs[k % 2], s2vs[k % 2]
            nch = EMB // LANES

            def body(i, carry):
                base = i * HIST
                accs = tuple(pbuf[base, pl.ds(c * LANES, LANES)]
                             for c in range(nch))

                def jblk(jb, accs):
                    row = base + 1 + jb * 7
                    for jj in range(7):
                        accs = tuple(
                            accs[c] + pbuf[row + jj, pl.ds(c * LANES, LANES)]
                            for c in range(nch))
                    return accs

                accs = lax.fori_loop(0, 7, jblk, accs)
                for c in range(nch):
                    s2v[i, pl.ds(c * LANES, LANES)] = accs[c]
                return carry

            lax.fori_loop(0, CHUNK, body, 0)

        def write_out(k):
            pltpu.async_copy(ubufs[k % 2],
                             ui_out.at[pl.ds(u0 + k * UROWS, UROWS), :], osem)
            pltpu.async_copy(pbufs[k % 2],
                             pref_out.at[pl.ds(p0 + k * PROWS, PROWS), :],
                             osem)
            pltpu.async_copy(s2vs[k % 2],
                             s2_out.at[pl.ds(b0 + k * CHUNK, CHUNK), :], osem)

        def drain_out(k):
            pltpu.make_async_copy(
                ubufs[k % 2],
                ui_out.at[pl.ds(u0 + k * UROWS, UROWS), :], osem).wait()
            pltpu.make_async_copy(
                pbufs[k % 2],
                pref_out.at[pl.ds(p0 + k * PROWS, PROWS), :], osem).wait()
            pltpu.make_async_copy(
                s2vs[k % 2],
                s2_out.at[pl.ds(b0 + k * CHUNK, CHUNK), :], osem).wait()

        stage_and_fire(0)
        for k in range(NCHUNK):
            if k + 1 < NCHUNK:
                if k >= 1:
                    drain_out(k - 1)
                stage_and_fire(k + 1)
            drain_gather(k)
            compute_s2(k)
            write_out(k)
        drain_out(NCHUNK - 2)
        drain_out(NCHUNK - 1)

    return sc_gather


_SC_GATHER = _build_sc_gather()

TCB = 512


def _tc_reduce_body(ui_ref, s2_ref, bias_ref, res_ref):
    ui = ui_ref[...]
    s2 = s2_ref[...]
    uin = ui[:, :EMB].reshape(TCB, 2, EMB)
    u1 = uin[:, 0, :]
    u2 = uin[:, 1, :]
    fm = u1 * u2 + (u1 + u2) * s2
    res_ref[...] = jnp.sum(fm, axis=1, keepdims=True) + bias_ref[0]


def _tc_reduce(raw_ui, s2, bias):
    return pl.pallas_call(
        _tc_reduce_body,
        grid=(B // TCB,),
        in_specs=[
            pl.BlockSpec((TCB * 2, ROWP), lambda i: (i, 0)),
            pl.BlockSpec((TCB, EMB), lambda i: (i, 0)),
            pl.BlockSpec(memory_space=pltpu.SMEM),
        ],
        out_specs=pl.BlockSpec((TCB, 1), lambda i: (i, 0)),
        out_shape=jax.ShapeDtypeStruct((B, 1), jnp.float32),
    )(raw_ui, s2, bias)


@jax.jit
def _fm(ui_pair, preference_index, ui_emb_w, feature_emb_w, bias):
    ui128 = _pad_table(ui_emb_w)
    feat128 = _pad_table(feature_emb_w)
    raw_ui, raw_pref, s2 = _SC_GATHER(
        ui_pair.astype(jnp.int32).reshape(-1),
        preference_index.astype(jnp.int32).reshape(-1),
        ui128, feat128)
    result = _tc_reduce(raw_ui, s2, bias)
    nz = jnp.concatenate(
        [raw_ui[:, :EMB].reshape(B, 2, EMB),
         raw_pref[:, :EMB].reshape(B, HIST, EMB)], axis=1)
    biascol = jnp.concatenate(
        [raw_ui[:, EMB:ROW].reshape(B, 2, 1),
         raw_pref[:, EMB:ROW].reshape(B, HIST, 1)], axis=1)
    return result, biascol, nz


def kernel(ui_pair, feature_index, preference_index, ui_emb_w, feature_emb_w,
           bias):
    del feature_index
    return _fm(ui_pair, preference_index, ui_emb_w, feature_emb_w, bias)
